# Initial kernel scaffold; baseline (speedup 1.0000x reference)
import jax
import jax.numpy as jnp
from jax.experimental import pallas as pl


def kernel(x, edge_index, edge_attr, WQ, WK, WE, WV):
    raise NotImplementedError("write your pallas kernel here")


# speedup vs baseline: 9.9364x; 9.9364x over previous
import functools

import numpy as np
import jax
import jax.numpy as jnp
from jax import lax
from jax.experimental import pallas as pl
from jax.experimental.pallas import tpu as pltpu
from jax.experimental.pallas import tpu_sc as plsc

_N = 10000
_E = 320000
_D = 128
_H = 8
_DH = 16

_NC = 2
_NS = 16
_NW = _NC * _NS
_BE = 64
_NBLKT = _E // _BE
_NBLKW = _NBLKT // _NW
_NBLKX = _NBLKT - _NBLKW * _NW
_RPT = 624
_TAIL = _N - _NS * _RPT

_f32 = jnp.float32
_i32 = jnp.int32


def _proj_body(x_ref, wkv_ref, wq_ref, kv_ref, q_ref):
    xb = x_ref[...]
    kv_ref[...] = jnp.dot(xb, wkv_ref[...], preferred_element_type=_f32)
    q_ref[...] = jnp.dot(xb, wq_ref[...], preferred_element_type=_f32) * 0.25


_proj = pl.pallas_call(
    _proj_body,
    grid=(10,),
    in_specs=[
        pl.BlockSpec((1000, _D), lambda i: (i, 0)),
        pl.BlockSpec((_D, 2 * _D), lambda i: (0, 0)),
        pl.BlockSpec((_D, _D), lambda i: (0, 0)),
    ],
    out_specs=[
        pl.BlockSpec((1000, 2 * _D), lambda i: (i, 0)),
        pl.BlockSpec((1000, _D), lambda i: (i, 0)),
    ],
    out_shape=[
        jax.ShapeDtypeStruct((_N, 2 * _D), _f32),
        jax.ShapeDtypeStruct((_N, _D), _f32),
    ],
)


def _ehm_body(ea_ref, we_ref, out_ref):
    out_ref[...] = jnp.dot(ea_ref[...], we_ref[...],
                           preferred_element_type=_f32)


_ehm = pl.pallas_call(
    _ehm_body,
    grid=(160,),
    in_specs=[
        pl.BlockSpec((2000, _DH), lambda i: (i, 0)),
        pl.BlockSpec((_DH, _D), lambda i: (0, 0)),
    ],
    out_specs=pl.BlockSpec((2000, _D), lambda i: (i, 0)),
    out_shape=jax.ShapeDtypeStruct((_E, _D), _f32),
)


def _sc_body(kv_hbm, q_hbm, eh_hbm, src_hbm, dst_hbm,
             wv_out, z_out,
             sidx, didx, kvrows, qrows, ehrows, zrow,
             wv_sh, z_sh, sem0, sem1):
    c = lax.axis_index("c")
    s = lax.axis_index("s")
    wid = s * _NC + c

    zeros16 = jnp.zeros((16,), _f32)

    def _zero_loc(r, carry):
        for cc in range(8):
            ehrows[r, pl.ds(cc * 16, 16)] = zeros16
        zrow[r, :] = zeros16
        return carry

    lax.fori_loop(0, _BE, _zero_loc, 0)

    start = s * _RPT
    for i in range(9):
        pltpu.sync_copy(ehrows, wv_sh.at[pl.ds(start + i * _BE, _BE), :])
        pltpu.sync_copy(zrow, z_sh.at[pl.ds(start + i * _BE, _BE), :])
    pltpu.sync_copy(ehrows.at[pl.ds(0, 48), :],
                    wv_sh.at[pl.ds(start + 576, 48), :])
    pltpu.sync_copy(zrow.at[pl.ds(0, 48), :],
                    z_sh.at[pl.ds(start + 576, 48), :])

    @pl.when(s == _NS - 1)
    def _zero_tail():
        pltpu.sync_copy(ehrows.at[pl.ds(0, _TAIL), :],
                        wv_sh.at[pl.ds(_NS * _RPT, _TAIL), :])
        pltpu.sync_copy(zrow.at[pl.ds(0, _TAIL), :],
                        z_sh.at[pl.ds(_NS * _RPT, _TAIL), :])

    plsc.subcore_barrier()

    iota16 = lax.iota(_i32, 16)
    ones16 = jnp.full((16,), 1, _i32)
    nblk = _NBLKW + jnp.where(wid < _NBLKX, 1, 0)

    def blk(b, carry):
        base = (wid + b * _NW) * _BE
        pltpu.sync_copy(src_hbm.at[pl.ds(base, _BE)], sidx)
        pltpu.sync_copy(dst_hbm.at[pl.ds(base, _BE)], didx)
        cp_kv = pltpu.async_copy(kv_hbm.at[sidx], kvrows, sem0)
        cp_q = pltpu.async_copy(q_hbm.at[didx], qrows, sem1)
        pltpu.sync_copy(eh_hbm.at[pl.ds(base, _BE), :], ehrows)
        cp_kv.wait()
        cp_q.wait()

        def grp(g, gc):
            el = g * 16
            rows = iota16 + el
            cv = jnp.zeros((16,), _i32)
            accs = []
            for h in range(_H):
                acc = None
                for d in range(_DH):
                    kc = plsc.load_gather(kvrows, [rows, cv])
                    qc = plsc.load_gather(qrows, [rows, cv])
                    ec = plsc.load_gather(ehrows, [rows, cv])
                    t = kc * qc * ec
                    acc = t if acc is None else acc + t
                    cv = cv + ones16
                accs.append(acc)
            svs = [jnp.exp(jnp.clip(a, -5.0, 5.0)) for a in accs]
            for h in range(_H):
                plsc.store_scatter(zrow, [rows, jnp.full((16,), h, _i32)],
                                   svs[h])
            cw = jnp.zeros((16,), _i32)
            for h in range(_H):
                for d in range(_DH):
                    vc = plsc.load_gather(kvrows, [rows, cv])
                    plsc.store_scatter(ehrows, [rows, cw], vc * svs[h])
                    cv = cv + ones16
                    cw = cw + ones16
            return gc

        lax.fori_loop(0, _BE // 16, grp, 0)

        pltpu.sync_copy(ehrows, wv_sh.at[didx], add=True)
        pltpu.sync_copy(zrow, z_sh.at[didx], add=True)
        return carry

    lax.fori_loop(0, nblk, blk, 0)

    plsc.subcore_barrier()
    pltpu.sync_copy(wv_sh.at[pl.ds(start, _RPT), :],
                    wv_out.at[c, pl.ds(start, _RPT), :])
    pltpu.sync_copy(z_sh.at[pl.ds(start, _RPT), :],
                    z_out.at[c, pl.ds(start, _RPT), :])

    @pl.when(s == _NS - 1)
    def _copy_tail():
        pltpu.sync_copy(wv_sh.at[pl.ds(_NS * _RPT, _TAIL), :],
                        wv_out.at[c, pl.ds(_NS * _RPT, _TAIL), :])
        pltpu.sync_copy(z_sh.at[pl.ds(_NS * _RPT, _TAIL), :],
                        z_out.at[c, pl.ds(_NS * _RPT, _TAIL), :])


_sc = functools.partial(
    pl.kernel,
    mesh=plsc.VectorSubcoreMesh(core_axis_name="c", subcore_axis_name="s"),
    compiler_params=pltpu.CompilerParams(
        use_tc_tiling_on_sc=False, needs_layout_passes=False),
    out_type=[
        jax.ShapeDtypeStruct((_NC, _N, _D), _f32),
        jax.ShapeDtypeStruct((_NC, _N, 16), _f32),
    ],
    scratch_types=[
        pltpu.VMEM((_BE,), _i32),
        pltpu.VMEM((_BE,), _i32),
        pltpu.VMEM((_BE, 2 * _D), _f32),
        pltpu.VMEM((_BE, _D), _f32),
        pltpu.VMEM((_BE, _D), _f32),
        pltpu.VMEM((_BE, 16), _f32),
        pltpu.VMEM_SHARED((_N, _D), _f32),
        pltpu.VMEM_SHARED((_N, 16), _f32),
        pltpu.SemaphoreType.DMA,
        pltpu.SemaphoreType.DMA,
    ],
)(_sc_body)


def _fin_body(wv_ref, z_ref, r_ref, o_ref):
    zs = z_ref[0] + z_ref[1]
    recip = 1.0 / (zs + 1e-6)
    zb = jnp.dot(recip, r_ref[...], preferred_element_type=_f32)
    o_ref[...] = (wv_ref[0] + wv_ref[1]) * zb


_fin = pl.pallas_call(
    _fin_body,
    grid=(10,),
    in_specs=[
        pl.BlockSpec((_NC, 1000, _D), lambda i: (0, i, 0)),
        pl.BlockSpec((_NC, 1000, 16), lambda i: (0, i, 0)),
        pl.BlockSpec((16, _D), lambda i: (0, 0)),
    ],
    out_specs=pl.BlockSpec((1000, _D), lambda i: (i, 0)),
    out_shape=jax.ShapeDtypeStruct((_N, _D), _f32),
)

_RNP = np.zeros((16, _D), np.float32)
for _h in range(_H):
    _RNP[_h, _h * _DH:(_h + 1) * _DH] = 1.0


def kernel(x, edge_index, edge_attr, WQ, WK, WE, WV):
    wkv = jnp.concatenate([WK, WV], axis=1)
    kv, q = _proj(x, wkv, WQ)
    eh = _ehm(edge_attr, WE)
    wv_p, z_p = _sc(kv, q, eh, edge_index[0], edge_index[1])
    return _fin(wv_p, z_p, jnp.asarray(_RNP))


# speedup vs baseline: 10.7491x; 1.0818x over previous
import functools

import numpy as np
import jax
import jax.numpy as jnp
from jax import lax
from jax.experimental import pallas as pl
from jax.experimental.pallas import tpu as pltpu
from jax.experimental.pallas import tpu_sc as plsc

_N = 10000
_E = 320000
_D = 128
_H = 8
_DH = 16

_NC = 2
_NS = 16
_NW = _NC * _NS
_EPW = _E // _NW
_BE = 40
_NBLK = _EPW // _BE
_NPAIR = _NBLK // 2
_RPT = 624
_TAIL = _N - _NS * _RPT

_f32 = jnp.float32
_i32 = jnp.int32


def _proj_body(x_ref, wkv_ref, wq_ref, kv_ref, q_ref):
    xb = x_ref[...]
    kv_ref[...] = jnp.dot(xb, wkv_ref[...], preferred_element_type=_f32)
    q_ref[...] = jnp.dot(xb, wq_ref[...], preferred_element_type=_f32) * 0.25


_proj = pl.pallas_call(
    _proj_body,
    grid=(10,),
    in_specs=[
        pl.BlockSpec((1000, _D), lambda i: (i, 0)),
        pl.BlockSpec((_D, 2 * _D), lambda i: (0, 0)),
        pl.BlockSpec((_D, _D), lambda i: (0, 0)),
    ],
    out_specs=[
        pl.BlockSpec((1000, 2 * _D), lambda i: (i, 0)),
        pl.BlockSpec((1000, _D), lambda i: (i, 0)),
    ],
    out_shape=[
        jax.ShapeDtypeStruct((_N, 2 * _D), _f32),
        jax.ShapeDtypeStruct((_N, _D), _f32),
    ],
)


def _ehm_body(ea_ref, we_ref, out_ref):
    out_ref[...] = jnp.dot(ea_ref[...], we_ref[...],
                           preferred_element_type=_f32)


_ehm = pl.pallas_call(
    _ehm_body,
    grid=(160,),
    in_specs=[
        pl.BlockSpec((2000, _DH), lambda i: (i, 0)),
        pl.BlockSpec((_DH, _D), lambda i: (0, 0)),
    ],
    out_specs=pl.BlockSpec((2000, _D), lambda i: (i, 0)),
    out_shape=jax.ShapeDtypeStruct((_E, _D), _f32),
)


def _sc_body(kv_hbm, q_hbm, eh_hbm, src_hbm, dst_hbm,
             wv_out, z_out,
             sidx0, didx0, kv0, q0, eh0, zr0,
             sidx1, didx1, kv1, q1, eh1, zr1,
             wv_sh, z_sh, semi, semg0, semg1, sems):
    c = lax.axis_index("c")
    s = lax.axis_index("s")
    wid = s * _NC + c
    w_e0 = wid * _EPW

    iota16 = lax.iota(_i32, 16)
    zeros16 = jnp.zeros((16,), _f32)
    ones16 = jnp.full((16,), 1, _i32)

    def _zero_eh(r, carry):
        for cc in range(8):
            eh0[r, pl.ds(cc * 16, 16)] = zeros16
        return carry

    lax.fori_loop(0, _BE, _zero_eh, 0)
    rz = iota16 >> 3
    cz = iota16 & 7
    for k in range(_BE // 2):
        plsc.store_scatter(zr0, [rz + 2 * k, cz], zeros16)

    start = s * _RPT
    for i in range(15):
        pltpu.sync_copy(eh0, wv_sh.at[pl.ds(start + i * _BE, _BE), :])
        pltpu.sync_copy(zr0, z_sh.at[pl.ds(start + i * _BE, _BE), :])
    pltpu.sync_copy(eh0.at[pl.ds(0, 24), :],
                    wv_sh.at[pl.ds(start + 600, 24), :])
    pltpu.sync_copy(zr0.at[pl.ds(0, 24), :],
                    z_sh.at[pl.ds(start + 600, 24), :])

    @pl.when(s == _NS - 1)
    def _zero_tail():
        pltpu.sync_copy(eh0.at[pl.ds(0, _TAIL), :],
                        wv_sh.at[pl.ds(_NS * _RPT, _TAIL), :])
        pltpu.sync_copy(zr0.at[pl.ds(0, _TAIL), :],
                        z_sh.at[pl.ds(_NS * _RPT, _TAIL), :])

    plsc.subcore_barrier()

    def _load_idx(base, si, di):
        c1 = pltpu.async_copy(src_hbm.at[pl.ds(base, _BE)], si, semi)
        c2 = pltpu.async_copy(dst_hbm.at[pl.ds(base, _BE)], di, semi)
        c1.wait()
        c2.wait()

    def _issue_gathers(base, si, di, kv, q, eh, semg):
        pltpu.async_copy(kv_hbm.at[si], kv, semg)
        pltpu.async_copy(q_hbm.at[di], q, semg)
        pltpu.async_copy(eh_hbm.at[pl.ds(base, _BE), :], eh, semg)

    def _drain_gathers(base, si, di, kv, q, eh, semg):
        pltpu.make_async_copy(kv_hbm.at[si], kv, semg).wait()
        pltpu.make_async_copy(q_hbm.at[di], q, semg).wait()
        pltpu.make_async_copy(eh_hbm.at[pl.ds(base, _BE), :], eh, semg).wait()

    def _compute(kv, q, eh, zr):
        def grp(g, carry):
            el = g * 16
            rows = iota16 + el
            mask = rows < _BE

            def head(h, counters):
                cv, cvv, cm = counters
                acc = None
                for d in range(_DH):
                    kc = plsc.load_gather(kv, [rows, cv], mask=mask)
                    qc = plsc.load_gather(q, [rows, cv], mask=mask)
                    ec = plsc.load_gather(eh, [rows, cv], mask=mask)
                    t = kc * qc * ec
                    acc = t if acc is None else acc + t
                    cv = cv + ones16
                sv = jnp.exp(jnp.clip(acc, -5.0, 5.0))
                hv = jnp.broadcast_to(h, (16,)).astype(_i32)
                plsc.store_scatter(zr, [rows, hv], sv, mask=mask)
                for d in range(_DH):
                    vc = plsc.load_gather(kv, [rows, cvv], mask=mask)
                    plsc.store_scatter(eh, [rows, cm], vc * sv, mask=mask)
                    cvv = cvv + ones16
                    cm = cm + ones16
                return (cv, cvv, cm)

            lax.fori_loop(
                0, _H, head,
                (jnp.zeros((16,), _i32),
                 jnp.full((16,), _D, _i32),
                 jnp.zeros((16,), _i32)))
            return carry

        lax.fori_loop(0, 3, grp, 0)

    def _scatter_sync(eh, zr, di):
        c1 = pltpu.async_copy(eh, wv_sh.at[di], sems, add=True)
        c2 = pltpu.async_copy(zr, z_sh.at[di], sems, add=True)
        c1.wait()
        c2.wait()

    _load_idx(w_e0, sidx0, didx0)
    _issue_gathers(w_e0, sidx0, didx0, kv0, q0, eh0, semg0)

    def pair(i, carry):
        base0 = w_e0 + i * (2 * _BE)
        base1 = base0 + _BE
        base2 = base0 + 2 * _BE
        _load_idx(base1, sidx1, didx1)
        _issue_gathers(base1, sidx1, didx1, kv1, q1, eh1, semg1)
        _drain_gathers(base0, sidx0, didx0, kv0, q0, eh0, semg0)
        _compute(kv0, q0, eh0, zr0)
        _scatter_sync(eh0, zr0, didx0)
        @pl.when(i < _NPAIR - 1)
        def _refill():
            _load_idx(base2, sidx0, didx0)
            _issue_gathers(base2, sidx0, didx0, kv0, q0, eh0, semg0)

        _drain_gathers(base1, sidx1, didx1, kv1, q1, eh1, semg1)
        _compute(kv1, q1, eh1, zr1)
        _scatter_sync(eh1, zr1, didx1)
        return carry

    lax.fori_loop(0, _NPAIR, pair, 0)

    plsc.subcore_barrier()
    pltpu.sync_copy(wv_sh.at[pl.ds(start, _RPT), :],
                    wv_out.at[c, pl.ds(start, _RPT), :])
    pltpu.sync_copy(z_sh.at[pl.ds(start, _RPT), :],
                    z_out.at[c, pl.ds(start, _RPT), :])

    @pl.when(s == _NS - 1)
    def _copy_tail():
        pltpu.sync_copy(wv_sh.at[pl.ds(_NS * _RPT, _TAIL), :],
                        wv_out.at[c, pl.ds(_NS * _RPT, _TAIL), :])
        pltpu.sync_copy(z_sh.at[pl.ds(_NS * _RPT, _TAIL), :],
                        z_out.at[c, pl.ds(_NS * _RPT, _TAIL), :])


_sc = functools.partial(
    pl.kernel,
    mesh=plsc.VectorSubcoreMesh(core_axis_name="c", subcore_axis_name="s"),
    compiler_params=pltpu.CompilerParams(
        use_tc_tiling_on_sc=False, needs_layout_passes=False),
    out_type=[
        jax.ShapeDtypeStruct((_NC, _N, _D), _f32),
        jax.ShapeDtypeStruct((_NC, _N, _H), _f32),
    ],
    scratch_types=[
        pltpu.VMEM((_BE,), _i32),
        pltpu.VMEM((_BE,), _i32),
        pltpu.VMEM((_BE, 2 * _D), _f32),
        pltpu.VMEM((_BE, _D), _f32),
        pltpu.VMEM((_BE, _D), _f32),
        pltpu.VMEM((_BE, _H), _f32),
        pltpu.VMEM((_BE,), _i32),
        pltpu.VMEM((_BE,), _i32),
        pltpu.VMEM((_BE, 2 * _D), _f32),
        pltpu.VMEM((_BE, _D), _f32),
        pltpu.VMEM((_BE, _D), _f32),
        pltpu.VMEM((_BE, _H), _f32),
        pltpu.VMEM_SHARED((_N, _D), _f32),
        pltpu.VMEM_SHARED((_N, _H), _f32),
        pltpu.SemaphoreType.DMA,
        pltpu.SemaphoreType.DMA,
        pltpu.SemaphoreType.DMA,
        pltpu.SemaphoreType.DMA,
    ],
)(_sc_body)


def _fin_body(wv_ref, z_ref, r_ref, o_ref):
    zs = z_ref[0] + z_ref[1]
    recip = 1.0 / (zs + 1e-6)
    zb = jnp.dot(recip, r_ref[...], preferred_element_type=_f32)
    o_ref[...] = (wv_ref[0] + wv_ref[1]) * zb


_fin = pl.pallas_call(
    _fin_body,
    grid=(10,),
    in_specs=[
        pl.BlockSpec((_NC, 1000, _D), lambda i: (0, i, 0)),
        pl.BlockSpec((_NC, 1000, _H), lambda i: (0, i, 0)),
        pl.BlockSpec((_H, _D), lambda i: (0, 0)),
    ],
    out_specs=pl.BlockSpec((1000, _D), lambda i: (i, 0)),
    out_shape=jax.ShapeDtypeStruct((_N, _D), _f32),
)

_RNP = np.kron(np.eye(_H), np.ones((1, _DH))).astype(np.float32)


def kernel(x, edge_index, edge_attr, WQ, WK, WE, WV):
    wkv = jnp.concatenate([WK, WV], axis=1)
    kv, q = _proj(x, wkv, WQ)
    eh = _ehm(edge_attr, WE)
    wv_p, z_p = _sc(kv, q, eh, edge_index[0], edge_index[1])
    return _fin(wv_p, z_p, jnp.asarray(_RNP))


# speedup vs baseline: 32.0241x; 2.9792x over previous
import functools

import numpy as np
import jax
import jax.numpy as jnp
from jax import lax
from jax.experimental import pallas as pl
from jax.experimental.pallas import tpu as pltpu
from jax.experimental.pallas import tpu_sc as plsc

_N = 10000
_E = 320000
_D = 128
_H = 8
_DH = 16

_NC = 2
_NS = 16
_NW = _NC * _NS
_EPW = _E // _NW
_BE = 40
_NBLK = _EPW // _BE
_NPAIR = _NBLK // 2
_RPT = 624
_TAIL = _N - _NS * _RPT

_f32 = jnp.float32
_i32 = jnp.int32


def _proj_body(x_ref, wkv_ref, wq_ref, kv_ref, q_ref):
    xb = x_ref[...]
    kv_ref[...] = jnp.dot(xb, wkv_ref[...], preferred_element_type=_f32)
    q_ref[...] = jnp.dot(xb, wq_ref[...], preferred_element_type=_f32) * 0.25


_proj = pl.pallas_call(
    _proj_body,
    grid=(10,),
    in_specs=[
        pl.BlockSpec((1000, _D), lambda i: (i, 0)),
        pl.BlockSpec((_D, 2 * _D), lambda i: (0, 0)),
        pl.BlockSpec((_D, _D), lambda i: (0, 0)),
    ],
    out_specs=[
        pl.BlockSpec((1000, 2 * _D), lambda i: (i, 0)),
        pl.BlockSpec((1000, _D), lambda i: (i, 0)),
    ],
    out_shape=[
        jax.ShapeDtypeStruct((_N, 2 * _D), _f32),
        jax.ShapeDtypeStruct((_N, _D), _f32),
    ],
)


def _ehm_body(ea_ref, we_ref, out_ref):
    out_ref[...] = jnp.dot(ea_ref[...], we_ref[...],
                           preferred_element_type=_f32)


_ehm = pl.pallas_call(
    _ehm_body,
    grid=(160,),
    in_specs=[
        pl.BlockSpec((2000, _DH), lambda i: (i, 0)),
        pl.BlockSpec((_DH, _D), lambda i: (0, 0)),
    ],
    out_specs=pl.BlockSpec((2000, _D), lambda i: (i, 0)),
    out_shape=jax.ShapeDtypeStruct((_E, _D), _f32),
)


def _sc_body(kv_hbm, q_hbm, eh_hbm, src_hbm, dst_hbm,
             wv_out, z_out,
             sidx0, didx0, kv0, q0, eh0, zr0,
             sidx1, didx1, kv1, q1, eh1, zr1,
             wv_sh, z_sh, semi, semg0, semg1, sems):
    c = lax.axis_index("c")
    s = lax.axis_index("s")
    wid = s * _NC + c
    w_e0 = wid * _EPW

    iota16 = lax.iota(_i32, 16)
    zeros16 = jnp.zeros((16,), _f32)
    ones16 = jnp.full((16,), 1, _i32)

    def _zero_eh(r, carry):
        for cc in range(8):
            eh0[r, pl.ds(cc * 16, 16)] = zeros16
        return carry

    lax.fori_loop(0, _BE, _zero_eh, 0)
    rz = iota16 >> 3
    cz = iota16 & 7
    for k in range(_BE // 2):
        plsc.store_scatter(zr0, [rz + 2 * k, cz], zeros16)

    start = s * _RPT
    for i in range(15):
        pltpu.sync_copy(eh0, wv_sh.at[pl.ds(start + i * _BE, _BE), :])
        pltpu.sync_copy(zr0, z_sh.at[pl.ds(start + i * _BE, _BE), :])
    pltpu.sync_copy(eh0.at[pl.ds(0, 24), :],
                    wv_sh.at[pl.ds(start + 600, 24), :])
    pltpu.sync_copy(zr0.at[pl.ds(0, 24), :],
                    z_sh.at[pl.ds(start + 600, 24), :])

    @pl.when(s == _NS - 1)
    def _zero_tail():
        pltpu.sync_copy(eh0.at[pl.ds(0, _TAIL), :],
                        wv_sh.at[pl.ds(_NS * _RPT, _TAIL), :])
        pltpu.sync_copy(zr0.at[pl.ds(0, _TAIL), :],
                        z_sh.at[pl.ds(_NS * _RPT, _TAIL), :])

    plsc.subcore_barrier()

    def _load_idx(base, si, di):
        c1 = pltpu.async_copy(src_hbm.at[pl.ds(base, _BE)], si, semi)
        c2 = pltpu.async_copy(dst_hbm.at[pl.ds(base, _BE)], di, semi)
        c1.wait()
        c2.wait()

    def _issue_gathers(base, si, di, kv, q, eh, semg):
        pltpu.async_copy(kv_hbm.at[si], kv, semg)
        pltpu.async_copy(q_hbm.at[di], q, semg)
        pltpu.async_copy(eh_hbm.at[pl.ds(base, _BE), :], eh, semg)

    def _drain_gathers(base, si, di, kv, q, eh, semg):
        pltpu.make_async_copy(kv_hbm.at[si], kv, semg).wait()
        pltpu.make_async_copy(q_hbm.at[di], q, semg).wait()
        pltpu.make_async_copy(eh_hbm.at[pl.ds(base, _BE), :], eh, semg).wait()

    fifteen16 = jnp.full((16,), 15, _i32)
    v128 = jnp.full((16,), _D, _i32)

    def _compute(kv, q, eh, zr):
        def grp(g, carry):
            el = g * 16
            rows = iota16 + el
            mask = rows < _BE

            def head(h, cb):
                rot = iota16
                acc = zeros16
                for d in range(_DH):
                    cv = cb | rot
                    kc = plsc.load_gather(kv, [rows, cv], mask=mask)
                    qc = plsc.load_gather(q, [rows, cv], mask=mask)
                    ec = plsc.load_gather(eh, [rows, cv], mask=mask)
                    acc = acc + kc * qc * ec
                    if d < _DH - 1:
                        rot = (rot + ones16) & fifteen16
                sv = jnp.exp(jnp.clip(acc, -5.0, 5.0))
                hv = jnp.broadcast_to(h, (16,)).astype(_i32)
                plsc.store_scatter(zr, [rows, hv], sv, mask=mask)
                rot = iota16
                for d in range(_DH):
                    cv = cb | rot
                    vc = plsc.load_gather(kv, [rows, cv + v128], mask=mask)
                    plsc.store_scatter(eh, [rows, cv], vc * sv, mask=mask)
                    if d < _DH - 1:
                        rot = (rot + ones16) & fifteen16
                return cb + _DH

            lax.fori_loop(0, _H, head, jnp.zeros((16,), _i32))
            return carry

        lax.fori_loop(0, 3, grp, 0)

    def _scatter_sync(eh, zr, di):
        c1 = pltpu.async_copy(eh, wv_sh.at[di], sems, add=True)
        c2 = pltpu.async_copy(zr, z_sh.at[di], sems, add=True)
        c1.wait()
        c2.wait()

    _load_idx(w_e0, sidx0, didx0)
    _issue_gathers(w_e0, sidx0, didx0, kv0, q0, eh0, semg0)

    def pair(i, carry):
        base0 = w_e0 + i * (2 * _BE)
        base1 = base0 + _BE
        base2 = base0 + 2 * _BE
        _load_idx(base1, sidx1, didx1)
        _issue_gathers(base1, sidx1, didx1, kv1, q1, eh1, semg1)
        _drain_gathers(base0, sidx0, didx0, kv0, q0, eh0, semg0)
        _compute(kv0, q0, eh0, zr0)
        _scatter_sync(eh0, zr0, didx0)
        @pl.when(i < _NPAIR - 1)
        def _refill():
            _load_idx(base2, sidx0, didx0)
            _issue_gathers(base2, sidx0, didx0, kv0, q0, eh0, semg0)

        _drain_gathers(base1, sidx1, didx1, kv1, q1, eh1, semg1)
        _compute(kv1, q1, eh1, zr1)
        _scatter_sync(eh1, zr1, didx1)
        return carry

    lax.fori_loop(0, _NPAIR, pair, 0)

    plsc.subcore_barrier()
    pltpu.sync_copy(wv_sh.at[pl.ds(start, _RPT), :],
                    wv_out.at[c, pl.ds(start, _RPT), :])
    pltpu.sync_copy(z_sh.at[pl.ds(start, _RPT), :],
                    z_out.at[c, pl.ds(start, _RPT), :])

    @pl.when(s == _NS - 1)
    def _copy_tail():
        pltpu.sync_copy(wv_sh.at[pl.ds(_NS * _RPT, _TAIL), :],
                        wv_out.at[c, pl.ds(_NS * _RPT, _TAIL), :])
        pltpu.sync_copy(z_sh.at[pl.ds(_NS * _RPT, _TAIL), :],
                        z_out.at[c, pl.ds(_NS * _RPT, _TAIL), :])


_sc = functools.partial(
    pl.kernel,
    mesh=plsc.VectorSubcoreMesh(core_axis_name="c", subcore_axis_name="s"),
    compiler_params=pltpu.CompilerParams(
        use_tc_tiling_on_sc=False, needs_layout_passes=False),
    out_type=[
        jax.ShapeDtypeStruct((_NC, _N, _D), _f32),
        jax.ShapeDtypeStruct((_NC, _N, _H), _f32),
    ],
    scratch_types=[
        pltpu.VMEM((_BE,), _i32),
        pltpu.VMEM((_BE,), _i32),
        pltpu.VMEM((_BE, 2 * _D), _f32),
        pltpu.VMEM((_BE, _D), _f32),
        pltpu.VMEM((_BE, _D), _f32),
        pltpu.VMEM((_BE, _H), _f32),
        pltpu.VMEM((_BE,), _i32),
        pltpu.VMEM((_BE,), _i32),
        pltpu.VMEM((_BE, 2 * _D), _f32),
        pltpu.VMEM((_BE, _D), _f32),
        pltpu.VMEM((_BE, _D), _f32),
        pltpu.VMEM((_BE, _H), _f32),
        pltpu.VMEM_SHARED((_N, _D), _f32),
        pltpu.VMEM_SHARED((_N, _H), _f32),
        pltpu.SemaphoreType.DMA,
        pltpu.SemaphoreType.DMA,
        pltpu.SemaphoreType.DMA,
        pltpu.SemaphoreType.DMA,
    ],
)(_sc_body)


def _fin_body(wv_ref, z_ref, r_ref, o_ref):
    zs = z_ref[0] + z_ref[1]
    recip = 1.0 / (zs + 1e-6)
    zb = jnp.dot(recip, r_ref[...], preferred_element_type=_f32)
    o_ref[...] = (wv_ref[0] + wv_ref[1]) * zb


_fin = pl.pallas_call(
    _fin_body,
    grid=(10,),
    in_specs=[
        pl.BlockSpec((_NC, 1000, _D), lambda i: (0, i, 0)),
        pl.BlockSpec((_NC, 1000, _H), lambda i: (0, i, 0)),
        pl.BlockSpec((_H, _D), lambda i: (0, 0)),
    ],
    out_specs=pl.BlockSpec((1000, _D), lambda i: (i, 0)),
    out_shape=jax.ShapeDtypeStruct((_N, _D), _f32),
)

_RNP = np.kron(np.eye(_H), np.ones((1, _DH))).astype(np.float32)


def kernel(x, edge_index, edge_attr, WQ, WK, WE, WV):
    wkv = jnp.concatenate([WK, WV], axis=1)
    kv, q = _proj(x, wkv, WQ)
    eh = _ehm(edge_attr, WE)
    wv_p, z_p = _sc(kv, q, eh, edge_index[0], edge_index[1])
    return _fin(wv_p, z_p, jnp.asarray(_RNP))
